# baseline (speedup 1.0000x reference)
import functools

import jax
import jax.numpy as jnp
from jax.experimental import pallas as pl
from jax.experimental.pallas import tpu as pltpu

_NCHUNK = 4


def _body(x_hbm, w1_ref, b1_ref, w2_ref, b2_ref, o_ref, buf, sems, *, inv_s, block_b, block_s, n_chunk):
    b = pl.program_id(0)

    def chunk_copy(i):
        return pltpu.make_async_copy(
            x_hbm.at[pl.ds(b * block_b, block_b), pl.ds(i * block_s, block_s), :],
            buf.at[:, pl.ds(i * block_s, block_s), :],
            sems.at[i],
        )

    for i in range(n_chunk):
        chunk_copy(i).start()

    acc = None
    for i in range(n_chunk):
        chunk_copy(i).wait()
        part = jnp.sum(buf[:, i * block_s:(i + 1) * block_s, :], axis=1)
        acc = part if acc is None else acc + part

    mean = acc * inv_s
    h = jnp.dot(mean, w1_ref[...], preferred_element_type=jnp.float32)
    h = jnp.maximum(h + b1_ref[...], 0.0)
    logits = jnp.dot(h, w2_ref[...], preferred_element_type=jnp.float32)
    logits = logits + b2_ref[...]
    m = jnp.max(logits, axis=-1, keepdims=True)
    z = logits - m
    o_ref[...] = z - jnp.log(jnp.sum(jnp.exp(z), axis=-1, keepdims=True))


def kernel(x, w1, b1, w2, b2):
    B, S, H = x.shape
    L = w2.shape[1]
    b1 = jnp.asarray(b1, jnp.float32).reshape(1, H)
    b2 = jnp.asarray(b2, jnp.float32).reshape(1, L)

    L_pad = -(-L // 128) * 128
    if L_pad != L:
        w2 = jnp.pad(w2, ((0, 0), (0, L_pad - L)))
        b2 = jnp.pad(b2, ((0, 0), (0, L_pad - L)), constant_values=-1e30)

    grid_b = 2 if B % 2 == 0 else 1
    block_b = B // grid_b
    block_s = S // _NCHUNK if S % _NCHUNK == 0 else S
    n_chunk = S // block_s

    body = functools.partial(
        _body, inv_s=1.0 / S, block_b=block_b, block_s=block_s, n_chunk=n_chunk
    )

    out = pl.pallas_call(
        body,
        out_shape=jax.ShapeDtypeStruct((B, L_pad), jnp.float32),
        grid=(grid_b,),
        in_specs=[
            pl.BlockSpec(memory_space=pl.ANY),
            pl.BlockSpec((H, H), lambda b: (0, 0)),
            pl.BlockSpec((1, H), lambda b: (0, 0)),
            pl.BlockSpec((H, L_pad), lambda b: (0, 0)),
            pl.BlockSpec((1, L_pad), lambda b: (0, 0)),
        ],
        out_specs=pl.BlockSpec((block_b, L_pad), lambda b: (b, 0)),
        scratch_shapes=[
            pltpu.VMEM((block_b, S, H), jnp.float32),
            pltpu.SemaphoreType.DMA((n_chunk,)),
        ],
        compiler_params=pltpu.CompilerParams(
            dimension_semantics=("parallel",),
            vmem_limit_bytes=60 * 1024 * 1024,
        ),
    )(x, w1, b1, w2, b2)

    return {"preds_length": out[:, :L]}
